# baseline (speedup 1.0000x reference)
import functools

import numpy as np
import jax
import jax.numpy as jnp
from jax import lax
from jax.experimental import pallas as pl
from jax.experimental.pallas import tpu as pltpu
from jax.experimental.pallas import tpu_sc as plsc

_MAX_LEN = 200
_HIDDEN = 64
_LANES = 16
_NBUF = 4
_NW = 32
_TW = 128


def _pos_encoding_np():
    pos = np.arange(_MAX_LEN, dtype=np.float32).reshape(-1, 1)
    div = np.power(
        10000.0, np.arange(0, _HIDDEN, 2, dtype=np.float32) / _HIDDEN
    )
    ang = pos / div
    P = np.zeros((_MAX_LEN, _HIDDEN), dtype=np.float32)
    P[:, 0::2] = np.sin(ang)
    P[:, 1::2] = np.cos(ang)
    return P


def _mesh():
    return plsc.VectorSubcoreMesh(core_axis_name="c", subcore_axis_name="s")


def _params(tc_tiling):
    return pltpu.CompilerParams(
        use_tc_tiling_on_sc=tc_tiling, needs_layout_passes=False
    )


@jax.jit
def _encoder_pipeline(xt, ts4, pe):
    H, V = ts4.shape
    L, B = xt.shape
    n_full = V // _TW
    quota = n_full // _NW
    n_extra = n_full - quota * _NW
    tail = V - n_full * _TW
    bpw = B // _NW

    @functools.partial(
        pl.kernel,
        mesh=_mesh(),
        compiler_params=_params(True),
        out_type=jax.ShapeDtypeStruct((V * H,), jnp.float32),
        scratch_types=[pltpu.VMEM((H, _TW), jnp.float32) for _ in range(2)]
        + [pltpu.VMEM((_TW * H,), jnp.float32) for _ in range(2)]
        + [pltpu.SemaphoreType.DMA for _ in range(4)],
    )
    def fmt(src, tail_hbm, dst, ta, tb, oa, ob, si0, si1, so0, so1):
        tins = [ta, tb]
        touts = [oa, ob]
        sem_i = [si0, si1]
        sem_o = [so0, so1]
        wid = lax.axis_index("s") * 2 + lax.axis_index("c")
        ihs = [
            (jnp.arange(_LANES, dtype=jnp.int32) + c0) * H
            for c0 in range(0, _TW, _LANES)
        ]

        def issue_in(t, b, w):
            pltpu.async_copy(
                src.at[:, pl.ds(t * _TW, w)],
                tins[b].at[:, pl.ds(0, w)], sem_i[b],
            )

        def wait_in(b, w):
            pltpu.make_async_copy(
                src.at[:, pl.ds(0, w)], tins[b].at[:, pl.ds(0, w)], sem_i[b]
            ).wait()

        def issue_out(t, b, w):
            pltpu.async_copy(
                touts[b].at[pl.ds(0, w * H)],
                dst.at[pl.ds(t * _TW * H, w * H)], sem_o[b],
            )

        def drain_out(b, w):
            pltpu.make_async_copy(
                dst.at[pl.ds(0, w * H)], touts[b].at[pl.ds(0, w * H)],
                sem_o[b],
            ).wait()

        def transpose(b, nslice):
            @pl.loop(0, H)
            def _(h):
                for kk in range(nslice):
                    v = tins[b][h, pl.ds(kk * _LANES, _LANES)]
                    plsc.store_scatter(touts[b], [ihs[kk] + h], v)

        base = wid * quota

        issue_in(base + 0, 0, _TW)
        issue_in(base + 1, 1, _TW)

        @pl.loop(0, quota)
        def _(i):
            for b in range(2):
                @pl.when(lax.rem(i, 2) == b)
                def _():
                    @pl.when(i >= 2)
                    def _():
                        drain_out(b, _TW)
                    wait_in(b, _TW)
                    transpose(b, _TW // _LANES)
                    issue_out(base + i, b, _TW)

                    @pl.when(i + 2 < quota)
                    def _():
                        issue_in(base + i + 2, b, _TW)

        drain_out(0, _TW)
        drain_out(1, _TW)

        @pl.when(wid < n_extra)
        def _():
            t = quota * _NW + wid
            issue_in(t, 0, _TW)
            wait_in(0, _TW)
            transpose(0, _TW // _LANES)
            issue_out(t, 0, _TW)
            drain_out(0, _TW)

        if tail:
            @pl.when(wid == n_extra)
            def _():
                pltpu.sync_copy(
                    tail_hbm, oa.at[pl.ds(0, tail * H)]
                )
                pltpu.sync_copy(
                    oa.at[pl.ds(0, tail * H)],
                    dst.at[pl.ds(n_full * _TW * H, tail * H)],
                )

    tail_rows = ts4[:, n_full * _TW:].T.reshape(-1)
    tflat = fmt(ts4, tail_rows)

    t2d_in = tflat.reshape(V, H)

    @functools.partial(
        pl.kernel,
        mesh=_mesh(),
        compiler_params=_params(False),
        out_type=jax.ShapeDtypeStruct((L * H * B,), jnp.float32),
        scratch_types=[
            pltpu.VMEM((L, bpw), jnp.int32),
            pltpu.VMEM((L * H,), jnp.float32),
        ]
        + [pltpu.VMEM((bpw, H), jnp.float32) for _ in range(_NBUF)]
        + [pltpu.VMEM((H * bpw,), jnp.float32) for _ in range(_NBUF)]
        + [pltpu.SemaphoreType.DMA for _ in range(2 * _NBUF)],
    )
    def enc(xt_hbm, t2d, pe_hbm, out_hbm, idx_v, pe_v,
            ga, gb, gc, gd, oa, ob, oc, od, sg0, sg1, sg2, sg3,
            sw0, sw1, sw2, sw3):
        grows = [ga, gb, gc, gd]
        outs = [oa, ob, oc, od]
        sem_g = [sg0, sg1, sg2, sg3]
        sem_w = [sw0, sw1, sw2, sw3]
        wid = lax.axis_index("s") * 2 + lax.axis_index("c")
        base = wid * bpw
        pltpu.sync_copy(pe_hbm, pe_v)
        pltpu.sync_copy(xt_hbm.at[:, pl.ds(base, bpw)], idx_v)

        ihs = [
            (jnp.arange(_LANES, dtype=jnp.int32) + h0) * bpw
            for h0 in range(0, H, _LANES)
        ]

        def issue_gather(l, b):
            pltpu.async_copy(t2d.at[idx_v.at[l]], grows[b], sem_g[b])

        def drain_g(b):
            pltpu.make_async_copy(
                t2d.at[pl.ds(0, bpw)], grows[b], sem_g[b]
            ).wait()

        def issue_write(l, b):
            for ht in range(H // 8):
                off = ((l * (H // 8) + ht) * (B // bpw) + wid) * (8 * bpw)
                pltpu.async_copy(
                    outs[b].at[pl.ds(ht * 8 * bpw, 8 * bpw)],
                    out_hbm.at[pl.ds(off, 8 * bpw)], sem_w[b],
                )

        def drain_w(b):
            pltpu.make_async_copy(
                out_hbm.at[pl.ds(0, H * bpw)], outs[b], sem_w[b]
            ).wait()

        def sweep(l, b):
            pek = [
                pe_v[pl.ds(l * H + h0, _LANES)]
                for h0 in range(0, H, _LANES)
            ]

            @pl.loop(0, bpw)
            def _(j):
                for kk in range(H // _LANES):
                    v = grows[b][j, pl.ds(kk * _LANES, _LANES)] + pek[kk]
                    plsc.store_scatter(outs[b], [ihs[kk] + j], v)

        def step(l, b, do_issue, do_drain_w):
            b2 = (b + 2) % _NBUF
            if do_drain_w:
                drain_w(b2)
            if do_issue:
                issue_gather(l + 2, b2)
            drain_g(b)
            sweep(l, b)
            issue_write(l, b)

        issue_gather(0, 0)
        issue_gather(1, 1)
        step(0, 0, True, False)
        step(1, 1, True, False)
        step(2, 2, True, True)
        step(3, 3, True, True)

        @pl.loop(1, L // _NBUF - 1)
        def _(g):
            l = g * _NBUF
            for b in range(_NBUF):
                step(l + b, b, True, True)

        l_last = L - _NBUF
        step(l_last + 0, 0, True, True)
        step(l_last + 1, 1, True, True)
        step(l_last + 2, 2, False, False)
        step(l_last + 3, 3, False, False)
        for b in range(_NBUF):
            drain_w(b)

    return enc(xt, t2d_in, pe)


def kernel(x, table):
    B, L = x.shape
    H = table.shape[1]
    xt = x.T.astype(jnp.int32)
    ts4 = table.T
    pe = jnp.asarray(_pos_encoding_np().reshape(-1))
    flat = _encoder_pipeline(xt, ts4, pe)
    o5 = flat.reshape(L, H // 8, B // 128, 8, 128)
    return o5.transpose(2, 4, 0, 1, 3).reshape(B, L, H)


# speedup vs baseline: 6.6742x; 6.6742x over previous
import functools

import numpy as np
import jax
import jax.numpy as jnp
from jax import lax
from jax.experimental import pallas as pl
from jax.experimental.pallas import tpu as pltpu
from jax.experimental.pallas import tpu_sc as plsc

_MAX_LEN = 200
_HIDDEN = 64
_LANES = 16
_NBUF = 4
_NW = 32
_TW = 128


def _pos_encoding_np():
    pos = np.arange(_MAX_LEN, dtype=np.float32).reshape(-1, 1)
    div = np.power(
        10000.0, np.arange(0, _HIDDEN, 2, dtype=np.float32) / _HIDDEN
    )
    ang = pos / div
    P = np.zeros((_MAX_LEN, _HIDDEN), dtype=np.float32)
    P[:, 0::2] = np.sin(ang)
    P[:, 1::2] = np.cos(ang)
    return P


def _mesh():
    return plsc.VectorSubcoreMesh(core_axis_name="c", subcore_axis_name="s")


def _params(tc_tiling):
    return pltpu.CompilerParams(
        use_tc_tiling_on_sc=tc_tiling, needs_layout_passes=False
    )


@jax.jit
def _encoder_pipeline(xt, ts4, pe):
    H, V = ts4.shape
    L, B = xt.shape
    n_full = V // _TW
    quota = n_full // _NW
    n_extra = n_full - quota * _NW
    tail = V - n_full * _TW
    bpw = B // _NW

    @functools.partial(
        pl.kernel,
        mesh=_mesh(),
        compiler_params=_params(True),
        out_type=jax.ShapeDtypeStruct((V * H,), jnp.float32),
        scratch_types=[pltpu.VMEM((H, _TW), jnp.float32) for _ in range(2)]
        + [pltpu.VMEM((_TW * H,), jnp.float32) for _ in range(2)]
        + [pltpu.SemaphoreType.DMA for _ in range(4)],
    )
    def fmt(src, tail_hbm, dst, ta, tb, oa, ob, si0, si1, so0, so1):
        tins = [ta, tb]
        touts = [oa, ob]
        sem_i = [si0, si1]
        sem_o = [so0, so1]
        wid = lax.axis_index("s") * 2 + lax.axis_index("c")
        ihs = [
            (jnp.arange(_LANES, dtype=jnp.int32) + c0) * H
            for c0 in range(0, _TW, _LANES)
        ]

        def issue_in(t, b, w):
            pltpu.async_copy(
                src.at[:, pl.ds(t * _TW, w)],
                tins[b].at[:, pl.ds(0, w)], sem_i[b],
            )

        def wait_in(b, w):
            pltpu.make_async_copy(
                src.at[:, pl.ds(0, w)], tins[b].at[:, pl.ds(0, w)], sem_i[b]
            ).wait()

        def issue_out(t, b, w):
            pltpu.async_copy(
                touts[b].at[pl.ds(0, w * H)],
                dst.at[pl.ds(t * _TW * H, w * H)], sem_o[b],
            )

        def drain_out(b, w):
            pltpu.make_async_copy(
                dst.at[pl.ds(0, w * H)], touts[b].at[pl.ds(0, w * H)],
                sem_o[b],
            ).wait()

        def transpose(b, nslice):
            @functools.partial(plsc.parallel_loop, 0, H, unroll=4)
            def _(h):
                for kk in range(nslice):
                    v = tins[b][h, pl.ds(kk * _LANES, _LANES)]
                    plsc.store_scatter(touts[b], [ihs[kk] + h], v)

        base = wid * quota

        issue_in(base + 0, 0, _TW)
        issue_in(base + 1, 1, _TW)

        @pl.loop(0, quota)
        def _(i):
            for b in range(2):
                @pl.when(lax.rem(i, 2) == b)
                def _():
                    @pl.when(i >= 2)
                    def _():
                        drain_out(b, _TW)
                    wait_in(b, _TW)
                    transpose(b, _TW // _LANES)
                    issue_out(base + i, b, _TW)

                    @pl.when(i + 2 < quota)
                    def _():
                        issue_in(base + i + 2, b, _TW)

        drain_out(0, _TW)
        drain_out(1, _TW)

        @pl.when(wid < n_extra)
        def _():
            t = quota * _NW + wid
            issue_in(t, 0, _TW)
            wait_in(0, _TW)
            transpose(0, _TW // _LANES)
            issue_out(t, 0, _TW)
            drain_out(0, _TW)

        if tail:
            @pl.when(wid == n_extra)
            def _():
                pltpu.sync_copy(
                    tail_hbm, oa.at[pl.ds(0, tail * H)]
                )
                pltpu.sync_copy(
                    oa.at[pl.ds(0, tail * H)],
                    dst.at[pl.ds(n_full * _TW * H, tail * H)],
                )

    tail_rows = ts4[:, n_full * _TW:].T.reshape(-1)
    tflat = fmt(ts4, tail_rows)

    t2d_in = tflat.reshape(V, H)

    @functools.partial(
        pl.kernel,
        mesh=_mesh(),
        compiler_params=_params(False),
        out_type=jax.ShapeDtypeStruct((L * H * B,), jnp.float32),
        scratch_types=[
            pltpu.VMEM((L, bpw), jnp.int32),
            pltpu.VMEM((L * H,), jnp.float32),
        ]
        + [pltpu.VMEM((bpw, H), jnp.float32) for _ in range(_NBUF)]
        + [pltpu.VMEM((H * bpw,), jnp.float32) for _ in range(_NBUF)]
        + [pltpu.SemaphoreType.DMA for _ in range(2 * _NBUF)],
    )
    def enc(xt_hbm, t2d, pe_hbm, out_hbm, idx_v, pe_v,
            ga, gb, gc, gd, oa, ob, oc, od, sg0, sg1, sg2, sg3,
            sw0, sw1, sw2, sw3):
        grows = [ga, gb, gc, gd]
        outs = [oa, ob, oc, od]
        sem_g = [sg0, sg1, sg2, sg3]
        sem_w = [sw0, sw1, sw2, sw3]
        wid = lax.axis_index("s") * 2 + lax.axis_index("c")
        base = wid * bpw
        pltpu.sync_copy(pe_hbm, pe_v)
        pltpu.sync_copy(xt_hbm.at[:, pl.ds(base, bpw)], idx_v)

        ihs = [
            (jnp.arange(_LANES, dtype=jnp.int32) + h0) * bpw
            for h0 in range(0, H, _LANES)
        ]

        def issue_gather(l, b):
            pltpu.async_copy(t2d.at[idx_v.at[l]], grows[b], sem_g[b])

        def drain_g(b):
            pltpu.make_async_copy(
                t2d.at[pl.ds(0, bpw)], grows[b], sem_g[b]
            ).wait()

        def issue_write(l, b):
            for ht in range(H // 8):
                off = ((l * (H // 8) + ht) * (B // bpw) + wid) * (8 * bpw)
                pltpu.async_copy(
                    outs[b].at[pl.ds(ht * 8 * bpw, 8 * bpw)],
                    out_hbm.at[pl.ds(off, 8 * bpw)], sem_w[b],
                )

        def drain_w(b):
            pltpu.make_async_copy(
                out_hbm.at[pl.ds(0, H * bpw)], outs[b], sem_w[b]
            ).wait()

        def sweep(l, b):
            pek = [
                pe_v[pl.ds(l * H + h0, _LANES)]
                for h0 in range(0, H, _LANES)
            ]

            @functools.partial(plsc.parallel_loop, 0, bpw, unroll=4)
            def _(j):
                for kk in range(H // _LANES):
                    v = grows[b][j, pl.ds(kk * _LANES, _LANES)] + pek[kk]
                    plsc.store_scatter(outs[b], [ihs[kk] + j], v)

        def step(l, b, do_issue, do_drain_w):
            b2 = (b + 2) % _NBUF
            if do_drain_w:
                drain_w(b2)
            if do_issue:
                issue_gather(l + 2, b2)
            drain_g(b)
            sweep(l, b)
            issue_write(l, b)

        issue_gather(0, 0)
        issue_gather(1, 1)
        step(0, 0, True, False)
        step(1, 1, True, False)
        step(2, 2, True, True)
        step(3, 3, True, True)

        @pl.loop(1, L // _NBUF - 1)
        def _(g):
            l = g * _NBUF
            for b in range(_NBUF):
                step(l + b, b, True, True)

        l_last = L - _NBUF
        step(l_last + 0, 0, True, True)
        step(l_last + 1, 1, True, True)
        step(l_last + 2, 2, False, False)
        step(l_last + 3, 3, False, False)
        for b in range(_NBUF):
            drain_w(b)

    return enc(xt, t2d_in, pe)


def kernel(x, table):
    B, L = x.shape
    H = table.shape[1]
    xt = x.T.astype(jnp.int32)
    ts4 = table.T
    pe = jnp.asarray(_pos_encoding_np().reshape(-1))
    flat = _encoder_pipeline(xt, ts4, pe)
    o5 = flat.reshape(L, H // 8, B // 128, 8, 128)
    return o5.transpose(2, 4, 0, 1, 3).reshape(B, L, H)
